# Initial kernel scaffold; baseline (speedup 1.0000x reference)
import jax
import jax.numpy as jnp
from jax import lax
from jax.experimental import pallas as pl
from jax.experimental.pallas import tpu as pltpu
from jax.experimental.pallas import tpu_sc as plsc

D_MODEL = 64
B = 4096
L = 200
N = B * L

_info = plsc.get_sparse_core_info()
NC = _info.num_cores
NS = _info.num_subcores
LANES = _info.num_lanes
NW = NC * NS

CHUNK = 256
PER_W = N // NW
N_CHUNKS = PER_W // CHUNK


def _sc_body(px_hbm, py_hbm, ex_hbm, ey_hbm, out_hbm,
             idx_x, idx_y, rows_x, rows_y, sem0, sem1):
  wid = lax.axis_index("s") * NC + lax.axis_index("c")
  w_base = wid * PER_W

  def chunk_body(c, _):
    base = w_base + c * CHUNK
    pltpu.sync_copy(px_hbm.at[pl.ds(base, CHUNK)], idx_x)
    pltpu.sync_copy(py_hbm.at[pl.ds(base, CHUNK)], idx_y)
    cpx = pltpu.async_copy(ex_hbm.at[idx_x], rows_x, sem0)
    cpy = pltpu.async_copy(ey_hbm.at[idx_y], rows_y, sem1)
    cpx.wait()
    cpy.wait()

    def add_body(i, _):
      for j in range(D_MODEL // LANES):
        sl = pl.ds(j * LANES, LANES)
        rows_x[i, sl] = rows_x[i, sl] + rows_y[i, sl]
      return 0

    lax.fori_loop(0, CHUNK, add_body, 0)
    pltpu.sync_copy(rows_x, out_hbm.at[pl.ds(base, CHUNK)])
    return 0

  lax.fori_loop(0, N_CHUNKS, chunk_body, 0)


_mesh = plsc.VectorSubcoreMesh(core_axis_name="c", subcore_axis_name="s")

_sc_kernel = pl.kernel(
    _sc_body,
    out_type=jax.ShapeDtypeStruct((N, D_MODEL), jnp.float32),
    mesh=_mesh,
    scratch_types=[
        pltpu.VMEM((CHUNK,), jnp.int32),
        pltpu.VMEM((CHUNK,), jnp.int32),
        pltpu.VMEM((CHUNK, D_MODEL), jnp.float32),
        pltpu.VMEM((CHUNK, D_MODEL), jnp.float32),
        pltpu.SemaphoreType.DMA,
        pltpu.SemaphoreType.DMA,
    ],
)


@jax.jit
def kernel(pos_x, pos_y, ex_weight, ey_weight):
  px = pos_x.reshape(N).astype(jnp.int32)
  py = pos_y.reshape(N).astype(jnp.int32)
  out = _sc_kernel(px, py, ex_weight, ey_weight)
  return out.reshape(B, L, D_MODEL)


# speedup vs baseline: 5.5863x; 5.5863x over previous
import jax
import jax.numpy as jnp
from jax import lax
from jax.experimental import pallas as pl
from jax.experimental.pallas import tpu as pltpu
from jax.experimental.pallas import tpu_sc as plsc

D_MODEL = 64
B = 4096
L = 200
N = B * L

_info = plsc.get_sparse_core_info()
NC = _info.num_cores
NS = _info.num_subcores
LANES = _info.num_lanes
NW = NC * NS

CHUNK = 256
PER_W = N // NW
N_CHUNKS = PER_W // CHUNK


def _sc_body(px_hbm, py_hbm, ex_hbm, ey_hbm, out_hbm,
             idx_x, idx_y, rows_x, rows_y, sem0, sem1):
  wid = lax.axis_index("s") * NC + lax.axis_index("c")
  w_base = wid * PER_W

  def chunk_body(c, _):
    base = w_base + c * CHUNK
    pltpu.sync_copy(px_hbm.at[pl.ds(base, CHUNK)], idx_x)
    pltpu.sync_copy(py_hbm.at[pl.ds(base, CHUNK)], idx_y)
    cpx = pltpu.async_copy(ex_hbm.at[idx_x], rows_x, sem0)
    cpy = pltpu.async_copy(ey_hbm.at[idx_y], rows_y, sem1)
    cpx.wait()
    cpy.wait()

    def add_body(i, _):
      for j in range(D_MODEL // LANES):
        sl = pl.ds(j * LANES, LANES)
        rows_x[i, sl] = rows_x[i, sl] + rows_y[i, sl]
      return 0

    lax.fori_loop(0, CHUNK, add_body, 0)
    pltpu.sync_copy(rows_x, out_hbm.at[pl.ds(base, CHUNK)])
    return 0

  lax.fori_loop(0, N_CHUNKS, chunk_body, 0)


_mesh = plsc.VectorSubcoreMesh(core_axis_name="c", subcore_axis_name="s")

_sc_kernel = pl.kernel(
    _sc_body,
    out_type=jax.ShapeDtypeStruct((N, D_MODEL), jnp.float32),
    mesh=_mesh,
    scratch_types=[
        pltpu.VMEM((CHUNK,), jnp.int32),
        pltpu.VMEM((CHUNK,), jnp.int32),
        pltpu.VMEM((CHUNK, D_MODEL), jnp.float32),
        pltpu.VMEM((CHUNK, D_MODEL), jnp.float32),
        pltpu.SemaphoreType.DMA,
        pltpu.SemaphoreType.DMA,
    ],
    compiler_params=pltpu.CompilerParams(use_tc_tiling_on_sc=False),
)


@jax.jit
def kernel(pos_x, pos_y, ex_weight, ey_weight):
  px = pos_x.reshape(N).astype(jnp.int32)
  py = pos_y.reshape(N).astype(jnp.int32)
  out = _sc_kernel(px, py, ex_weight, ey_weight)
  return out.reshape(B, L, D_MODEL)
